# baseline (speedup 1.0000x reference)
import jax
import jax.numpy as jnp
from jax import lax
from jax.experimental import pallas as pl
from jax.experimental.pallas import tpu as pltpu
from jax.experimental.pallas import tpu_sc as plsc

_TEMP = 1.0


def _proj_body(e_ref, wt_ref, b_ref, o_ref):
    o_ref[...] = (
        jnp.dot(e_ref[...], wt_ref[...], preferred_element_type=jnp.float32)
        + b_ref[...]
    )


def _project(e2, Wt, b2, N, D):
    rows = 2048
    return pl.pallas_call(
        _proj_body,
        grid=(N // rows,),
        in_specs=[
            pl.BlockSpec((rows, D), lambda i: (i, 0)),
            pl.BlockSpec((D, D), lambda i: (0, 0)),
            pl.BlockSpec((1, D), lambda i: (0, 0)),
        ],
        out_specs=pl.BlockSpec((rows, D), lambda i: (i, 0)),
        out_shape=jax.ShapeDtypeStruct((N, D), jnp.float32),
    )(e2, Wt, b2)


def _widen_body(t_ref, o_ref):
    o_ref[:, 0:64] = t_ref[...]


def _widen_table(table, V, D):
    rows = 4000
    return pl.pallas_call(
        _widen_body,
        grid=(V // rows,),
        in_specs=[pl.BlockSpec((rows, D), lambda i: (i, 0))],
        out_specs=pl.BlockSpec((rows, 2 * D), lambda i: (i, 0)),
        out_shape=jax.ShapeDtypeStruct((V, 2 * D), jnp.float32),
    )(table)


def _make_scores(N, D, NEG):
    NC, NS = 2, 16
    NW = NC * NS
    RW = N // NW
    R = 4
    NCH = RW // R
    L = 16
    JG = NEG // L
    IW = 72
    G = 1 + NEG
    mesh = plsc.VectorSubcoreMesh(
        core_axis_name="c", subcore_axis_name="s", num_cores=NC, num_subcores=NS
    )

    def body(x_hbm, idxc_hbm, table_hbm, pos_hbm, neg_hbm,
             idx_all, comb_v, x_v, nout_v, pout_all,
             sem_in0, sem_in1, sem_out0, sem_out1):
        wid = lax.axis_index("s") * NC + lax.axis_index("c")
        base = wid * RW
        pltpu.sync_copy(idxc_hbm.at[pl.ds(base * IW, RW * IW)], idx_all)

        sems_in = (sem_in0, sem_in1)
        sems_out = (sem_out0, sem_out1)
        lane = jnp.arange(L, dtype=jnp.int32)

        def in_copies(gg, b):
            lr = gg * R
            sem = sems_in[b]
            cps = [
                pltpu.make_async_copy(
                    x_hbm.at[pl.ds((base + lr) * D, R * D)], x_v.at[b], sem),
            ]
            for r in range(R):
                cps.append(pltpu.make_async_copy(
                    table_hbm.at[idx_all.at[pl.ds((lr + r) * IW, G)]],
                    comb_v.at[b, r, pl.ds(0, G)], sem))
            return cps

        def out_copies(gg, b):
            return [
                pltpu.make_async_copy(
                    nout_v.at[b],
                    neg_hbm.at[pl.ds((base + gg * R) * NEG, R * NEG)],
                    sems_out[b]),
            ]

        def fire(gg, b):
            for c in in_copies(gg, b):
                c.start()

        fire(0, 0)
        fire(1, 1)

        def tbody(t, pos_acc):
            for b in range(2):
                gg = t * 2 + b
                for c in in_copies(gg, b):
                    c.wait()

                @pl.when(gg >= 2)
                def _():
                    for c in out_copies(gg - 2, b):
                        c.wait()

                for r in range(R):
                    x0 = x_v[b, r * D + 0:r * D + 16]
                    x1 = x_v[b, r * D + 16:r * D + 32]
                    x2 = x_v[b, r * D + 32:r * D + 48]
                    x3 = x_v[b, r * D + 48:r * D + 64]
                    ps = (comb_v[b, r, 0, 0:16] * x0
                          + comb_v[b, r, 0, 16:32] * x1
                          + comb_v[b, r, 0, 32:48] * x2
                          + comb_v[b, r, 0, 48:64] * x3)
                    ln = 8 * (t % 2) + 4 * b + r
                    pos_acc = pos_acc + jnp.sum(ps) * jnp.where(
                        lane == ln, 1.0, 0.0).astype(jnp.float32)

                    bi = jnp.full((L,), b, jnp.int32)
                    ri = jnp.full((L,), r, jnp.int32)

                    def gbody(jg, c2, b=b, r=r, bi=bi, ri=ri,
                              x0=x0, x1=x1, x2=x2, x3=x3):
                        acc = jnp.zeros((L,), jnp.float32)
                        rb = jg * L + 1
                        for jj in range(L):
                            row_i = jnp.full((L,), rb + jj, jnp.int32)
                            a = (plsc.load_gather(
                                     comb_v, [bi, ri, row_i, lane]) * x0
                                 + plsc.load_gather(
                                     comb_v, [bi, ri, row_i, lane + 16]) * x1
                                 + plsc.load_gather(
                                     comb_v, [bi, ri, row_i, lane + 32]) * x2
                                 + plsc.load_gather(
                                     comb_v, [bi, ri, row_i, lane + 48]) * x3)
                            oh = jnp.where(lane == jj, 1.0, 0.0).astype(
                                jnp.float32)
                            acc = acc + jnp.sum(a) * oh
                        plsc.store_scatter(
                            nout_v, [bi, r * NEG + jg * L + lane], acc)
                        return c2

                    lax.fori_loop(0, JG, gbody, 0)

                for c in out_copies(gg, b):
                    c.start()

                @pl.when(gg + 2 < NCH)
                def _():
                    fire(gg + 2, b)

            @pl.when(t % 2 == 1)
            def _():
                o = (t // 2) * L + lane
                plsc.store_scatter(pout_all, [o // 128, o % 128], pos_acc)

            return jnp.where(t % 2 == 1, jnp.zeros((L,), jnp.float32),
                             pos_acc)

        lax.fori_loop(0, NCH // 2, tbody, jnp.zeros((L,), jnp.float32))
        for c in out_copies(NCH - 2, 0):
            c.wait()
        for c in out_copies(NCH - 1, 1):
            c.wait()
        pltpu.sync_copy(pout_all, pos_hbm.at[wid])

    return pl.kernel(
        body,
        out_type=[
            jax.ShapeDtypeStruct((NW, RW // 128, 128), jnp.float32),
            jax.ShapeDtypeStruct((N * NEG,), jnp.float32),
        ],
        mesh=mesh,
        compiler_params=pltpu.CompilerParams(
            needs_layout_passes=False, use_tc_tiling_on_sc=True),
        scratch_types=[
            pltpu.VMEM((RW * IW,), jnp.int32),
            pltpu.VMEM((2, R, IW, 2 * D), jnp.float32),
            pltpu.VMEM((2, R * D), jnp.float32),
            pltpu.VMEM((2, R * NEG), jnp.float32),
            pltpu.VMEM((RW // 128, 128), jnp.float32),
            pltpu.SemaphoreType.DMA,
            pltpu.SemaphoreType.DMA,
            pltpu.SemaphoreType.DMA,
            pltpu.SemaphoreType.DMA,
        ],
    )


def _make_loss(N, NEG):
    rows = 2048
    inv_t = 1.0 / _TEMP

    def body(pos_ref, neg_ref, logits_ref, loss_ref):
        i = pl.program_id(0)
        pos = pos_ref[...]
        neg = neg_ref[...]
        logits_ref[...] = jnp.concatenate([pos, neg], axis=1)
        sp = pos * inv_t
        sn = neg * inv_t
        m = jnp.maximum(sp, jnp.max(sn, axis=1, keepdims=True))
        lse = m + jnp.log(
            jnp.exp(sp - m) + jnp.sum(jnp.exp(sn - m), axis=1, keepdims=True))
        c = jnp.sum(lse - sp)

        @pl.when(i == 0)
        def _():
            loss_ref[0, 0] = 0.0

        loss_ref[0, 0] += c / N

    return pl.pallas_call(
        body,
        grid=(N // rows,),
        in_specs=[
            pl.BlockSpec((rows, 1), lambda i: (i, 0)),
            pl.BlockSpec((rows, NEG), lambda i: (i, 0)),
        ],
        out_specs=[
            pl.BlockSpec((rows, 1 + NEG), lambda i: (i, 0)),
            pl.BlockSpec((1, 1), lambda i: (0, 0), memory_space=pltpu.SMEM),
        ],
        out_shape=[
            jax.ShapeDtypeStruct((N, 1 + NEG), jnp.float32),
            jax.ShapeDtypeStruct((1, 1), jnp.float32),
        ],
    )


def kernel(embeds, labels, table, W, b, neg_samples):
    B, T, D = embeds.shape
    N = B * T
    V = table.shape[0]
    NEG = neg_samples.shape[1]

    e2 = embeds.reshape(N, D)
    x = _project(e2, W.T, b.reshape(1, D), N, D)
    table_w = _widen_table(table, V, D)
    idxc = jnp.concatenate(
        [labels.reshape(N, 1), neg_samples,
         jnp.zeros((N, 7), jnp.int32)], axis=1).reshape(-1)
    pos3, neg1 = _make_scores(N, D, NEG)(x.reshape(-1), idxc, table_w)
    pos = pos3.reshape(N, 1)
    neg = neg1.reshape(N, NEG)
    logits, loss = _make_loss(N, NEG)(pos, neg)
    return logits, loss.reshape(())


# speedup vs baseline: 1.0770x; 1.0770x over previous
import numpy as np
import jax
import jax.numpy as jnp
from jax import lax
from jax.experimental import pallas as pl
from jax.experimental.pallas import tpu as pltpu
from jax.experimental.pallas import tpu_sc as plsc

_TEMP = 1.0

_PERM = np.concatenate([
    np.arange(0, 32, 2), np.arange(1, 32, 2),
    np.arange(32, 64, 2), np.arange(33, 64, 2),
])


def _proj_body(e_ref, wt_ref, b_ref, o_ref):
    o_ref[...] = (
        jnp.dot(e_ref[...], wt_ref[...], preferred_element_type=jnp.float32)
        + b_ref[...]
    )


def _project(e2, Wt, b2, N, D):
    rows = 2048
    return pl.pallas_call(
        _proj_body,
        grid=(N // rows,),
        in_specs=[
            pl.BlockSpec((rows, D), lambda i: (i, 0)),
            pl.BlockSpec((D, D), lambda i: (0, 0)),
            pl.BlockSpec((1, D), lambda i: (0, 0)),
        ],
        out_specs=pl.BlockSpec((rows, D), lambda i: (i, 0)),
        out_shape=jax.ShapeDtypeStruct((N, D), jnp.float32),
    )(e2, Wt, b2)


def _make_scores(N, D, NEG):
    NC, NS = 2, 16
    NW = NC * NS
    RW = N // NW
    R = 8
    NCH = RW // R
    L = 16
    JG = NEG // L
    mesh = plsc.VectorSubcoreMesh(
        core_axis_name="c", subcore_axis_name="s", num_cores=NC, num_subcores=NS
    )
    ilv = plsc.PackFormat.INTERLEAVED

    def body(x_hbm, lab_hbm, idx_hbm, table_hbm, pos_hbm, neg_hbm,
             idx_all, lab_all, x_v, pos_v, neg_v, nout_v, pout_v,
             sem_in0, sem_in1, sem_out0, sem_out1):
        wid = lax.axis_index("s") * NC + lax.axis_index("c")
        base = wid * RW
        pltpu.sync_copy(idx_hbm.at[pl.ds(base, RW)], idx_all)
        pltpu.sync_copy(lab_hbm.at[pl.ds(base, RW)], lab_all)

        sems_in = (sem_in0, sem_in1)
        sems_out = (sem_out0, sem_out1)
        lane = jnp.arange(L, dtype=jnp.int32)

        def in_copies(gg, b):
            lr = gg * R
            sem = sems_in[b]
            cps = [
                pltpu.make_async_copy(
                    x_hbm.at[pl.ds(base + lr, R)], x_v.at[b], sem),
                pltpu.make_async_copy(
                    table_hbm.at[lab_all.at[pl.ds(lr, R)]], pos_v.at[b], sem),
            ]
            for r in range(R):
                cps.append(pltpu.make_async_copy(
                    table_hbm.at[idx_all.at[lr + r]], neg_v.at[b, r], sem))
            return cps

        def out_copies(gg, b):
            return [
                pltpu.make_async_copy(
                    nout_v.at[b], neg_hbm.at[pl.ds(base + gg * R, R)],
                    sems_out[b]),
                pltpu.make_async_copy(
                    pout_v.at[b], pos_hbm.at[wid * NCH + gg], sems_out[b]),
            ]

        def fire(gg, b):
            for c in in_copies(gg, b):
                c.start()

        fire(0, 0)
        fire(1, 1)

        def dot4(xp0, xp1, xp2, xp3, g0, g1):
            e0, o0 = plsc.unpack(g0, format=ilv)
            e1, o1 = plsc.unpack(g1, format=ilv)
            return e0 * xp0 + o0 * xp1 + e1 * xp2 + o1 * xp3

        def tbody(t, carry):
            for b in range(2):
                gg = t * 2 + b
                for c in in_copies(gg, b):
                    c.wait()

                @pl.when(gg >= 2)
                def _():
                    for c in out_copies(gg - 2, b):
                        c.wait()

                def rbody(r, pos_acc, b=b):
                    xp0 = x_v[b, r, 0:16]
                    xp1 = x_v[b, r, 16:32]
                    xp2 = x_v[b, r, 32:48]
                    xp3 = x_v[b, r, 48:64]
                    ps = dot4(xp0, xp1, xp2, xp3,
                              pos_v[b, r, 0:32], pos_v[b, r, 32:64])
                    pos_acc = pos_acc + jnp.sum(ps) * jnp.where(
                        lane == r, 1.0, 0.0).astype(jnp.float32)

                    def gbody(jg, c2, b=b, r=r,
                              xp0=xp0, xp1=xp1, xp2=xp2, xp3=xp3):
                        acc = jnp.zeros((L,), jnp.float32)
                        for jj in range(L):
                            j = jg * L + jj
                            a = dot4(xp0, xp1, xp2, xp3,
                                     neg_v[b, r, j, 0:32],
                                     neg_v[b, r, j, 32:64])
                            oh = jnp.where(lane == jj, 1.0, 0.0).astype(
                                jnp.float32)
                            acc = acc + jnp.sum(a) * oh
                        nout_v[b, r, pl.ds(jg * L, L)] = acc
                        return c2

                    lax.fori_loop(0, JG, gbody, 0)
                    return pos_acc

                pos_acc = lax.fori_loop(
                    0, R, rbody, jnp.zeros((L,), jnp.float32))
                pout_v[b, :] = pos_acc

                for c in out_copies(gg, b):
                    c.start()

                @pl.when(gg + 2 < NCH)
                def _():
                    fire(gg + 2, b)
            return carry

        lax.fori_loop(0, NCH // 2, tbody, 0)
        for c in out_copies(NCH - 2, 0):
            c.wait()
        for c in out_copies(NCH - 1, 1):
            c.wait()

    return pl.kernel(
        body,
        out_type=[
            jax.ShapeDtypeStruct((NW * NCH, L), jnp.float32),
            jax.ShapeDtypeStruct((N, NEG), jnp.float32),
        ],
        mesh=mesh,
        compiler_params=pltpu.CompilerParams(
            needs_layout_passes=False, use_tc_tiling_on_sc=False),
        scratch_types=[
            pltpu.VMEM((RW, NEG), jnp.int32),
            pltpu.VMEM((RW,), jnp.int32),
            pltpu.VMEM((2, R, D), jnp.float32),
            pltpu.VMEM((2, R, D), jnp.bfloat16),
            pltpu.VMEM((2, R, NEG, D), jnp.bfloat16),
            pltpu.VMEM((2, R, NEG), jnp.float32),
            pltpu.VMEM((2, L), jnp.float32),
            pltpu.SemaphoreType.DMA,
            pltpu.SemaphoreType.DMA,
            pltpu.SemaphoreType.DMA,
            pltpu.SemaphoreType.DMA,
        ],
    )


def _make_loss(N, NEG):
    rows = 2048
    inv_t = 1.0 / _TEMP

    def body(pos_ref, neg_ref, logits_ref, loss_ref):
        i = pl.program_id(0)
        pos = pos_ref[...]
        neg = neg_ref[...]
        logits_ref[...] = jnp.concatenate([pos, neg], axis=1)
        sp = pos * inv_t
        sn = neg * inv_t
        m = jnp.maximum(sp, jnp.max(sn, axis=1, keepdims=True))
        lse = m + jnp.log(
            jnp.exp(sp - m) + jnp.sum(jnp.exp(sn - m), axis=1, keepdims=True))
        c = jnp.sum(lse - sp)

        @pl.when(i == 0)
        def _():
            loss_ref[0, 0] = 0.0

        loss_ref[0, 0] += c / N

    return pl.pallas_call(
        body,
        grid=(N // rows,),
        in_specs=[
            pl.BlockSpec((rows, 1), lambda i: (i, 0)),
            pl.BlockSpec((rows, NEG), lambda i: (i, 0)),
        ],
        out_specs=[
            pl.BlockSpec((rows, 1 + NEG), lambda i: (i, 0)),
            pl.BlockSpec((1, 1), lambda i: (0, 0), memory_space=pltpu.SMEM),
        ],
        out_shape=[
            jax.ShapeDtypeStruct((N, 1 + NEG), jnp.float32),
            jax.ShapeDtypeStruct((1, 1), jnp.float32),
        ],
    )


def kernel(embeds, labels, table, W, b, neg_samples):
    B, T, D = embeds.shape
    N = B * T
    NEG = neg_samples.shape[1]

    e2 = embeds.reshape(N, D)
    x = _project(e2, W.T, b.reshape(1, D), N, D)
    xp = x[:, _PERM]
    tbl16 = table.astype(jnp.bfloat16)
    pos2, neg = _make_scores(N, D, NEG)(
        xp, labels.reshape(N), neg_samples, tbl16)
    R = 8
    pos = pos2[:, :R].reshape(N, 1)
    logits, loss = _make_loss(N, NEG)(pos, neg)
    return logits, loss.reshape(())


# speedup vs baseline: 1.2309x; 1.1429x over previous
import jax
import jax.numpy as jnp
from jax import lax
from jax.experimental import pallas as pl
from jax.experimental.pallas import tpu as pltpu
from jax.experimental.pallas import tpu_sc as plsc

_TEMP = 1.0


def _proj_body(e_ref, wt_ref, b_ref, o_ref):
    o_ref[...] = (
        jnp.dot(e_ref[...], wt_ref[...], preferred_element_type=jnp.float32)
        + b_ref[...]
    )


def _project(e2, Wt, b2, N, D):
    rows = 2048
    return pl.pallas_call(
        _proj_body,
        grid=(N // rows,),
        in_specs=[
            pl.BlockSpec((rows, D), lambda i: (i, 0)),
            pl.BlockSpec((D, D), lambda i: (0, 0)),
            pl.BlockSpec((1, D), lambda i: (0, 0)),
        ],
        out_specs=pl.BlockSpec((rows, D), lambda i: (i, 0)),
        out_shape=jax.ShapeDtypeStruct((N, D), jnp.float32),
    )(e2, Wt, b2)


def _widen_body(t_ref, o_ref):
    o_ref[:, 0:64] = t_ref[...].T


def _widen_table(table_t, V, D):
    rows = 2048
    return pl.pallas_call(
        _widen_body,
        grid=((V + rows - 1) // rows,),
        in_specs=[pl.BlockSpec((D, rows), lambda i: (0, i))],
        out_specs=pl.BlockSpec((rows, 2 * D), lambda i: (i, 0)),
        out_shape=jax.ShapeDtypeStruct((V, 2 * D), jnp.float32),
    )(table_t)


def _make_scores(N, D, NEG):
    NC, NS = 2, 16
    NW = NC * NS
    RW = N // NW
    R = 4
    NCH = RW // R
    L = 16
    JG = NEG // L
    IW = 72
    G = 1 + NEG
    mesh = plsc.VectorSubcoreMesh(
        core_axis_name="c", subcore_axis_name="s", num_cores=NC, num_subcores=NS
    )

    def body(x_hbm, idxc_hbm, table_hbm, pos_hbm, neg_hbm,
             idx_all, comb_v, x_v, nout_v, pout_all,
             sem_in0, sem_in1, sem_out0, sem_out1):
        wid = lax.axis_index("s") * NC + lax.axis_index("c")
        base = wid * RW
        pltpu.sync_copy(idxc_hbm.at[pl.ds(base * IW, RW * IW)], idx_all)

        sems_in = (sem_in0, sem_in1)
        sems_out = (sem_out0, sem_out1)
        lane = jnp.arange(L, dtype=jnp.int32)

        def in_copies(gg, b):
            lr = gg * R
            sem = sems_in[b]
            cps = [
                pltpu.make_async_copy(
                    x_hbm.at[pl.ds((base + lr) * D, R * D)], x_v.at[b], sem),
            ]
            for r in range(R):
                cps.append(pltpu.make_async_copy(
                    table_hbm.at[idx_all.at[pl.ds((lr + r) * IW, G)]],
                    comb_v.at[b, r, pl.ds(0, G)], sem))
            return cps

        def out_copies(gg, b):
            return [
                pltpu.make_async_copy(
                    nout_v.at[b],
                    neg_hbm.at[pl.ds((base + gg * R) * NEG, R * NEG)],
                    sems_out[b]),
            ]

        def fire(gg, b):
            for c in in_copies(gg, b):
                c.start()

        fire(0, 0)
        fire(1, 1)

        def tbody(t, pos_acc):
            for b in range(2):
                gg = t * 2 + b
                for c in in_copies(gg, b):
                    c.wait()

                @pl.when(gg >= 2)
                def _():
                    for c in out_copies(gg - 2, b):
                        c.wait()

                for r in range(R):
                    x0 = x_v[b, r * D + 0:r * D + 16]
                    x1 = x_v[b, r * D + 16:r * D + 32]
                    x2 = x_v[b, r * D + 32:r * D + 48]
                    x3 = x_v[b, r * D + 48:r * D + 64]
                    ps = (comb_v[b, r, 0, 0:16] * x0
                          + comb_v[b, r, 0, 16:32] * x1
                          + comb_v[b, r, 0, 32:48] * x2
                          + comb_v[b, r, 0, 48:64] * x3)
                    ln = 8 * (t % 2) + 4 * b + r
                    pos_acc = pos_acc + jnp.sum(ps) * jnp.where(
                        lane == ln, 1.0, 0.0).astype(jnp.float32)

                    bi = jnp.full((L,), b, jnp.int32)
                    ri = jnp.full((L,), r, jnp.int32)

                    def gbody(jg, c2, b=b, r=r, bi=bi, ri=ri,
                              x0=x0, x1=x1, x2=x2, x3=x3):
                        acc = jnp.zeros((L,), jnp.float32)
                        rb = jg * L + 1
                        for jj in range(L):
                            row_i = jnp.full((L,), rb + jj, jnp.int32)
                            a = (plsc.load_gather(
                                     comb_v, [bi, ri, row_i, lane]) * x0
                                 + plsc.load_gather(
                                     comb_v, [bi, ri, row_i, lane + 16]) * x1
                                 + plsc.load_gather(
                                     comb_v, [bi, ri, row_i, lane + 32]) * x2
                                 + plsc.load_gather(
                                     comb_v, [bi, ri, row_i, lane + 48]) * x3)
                            oh = jnp.where(lane == jj, 1.0, 0.0).astype(
                                jnp.float32)
                            acc = acc + jnp.sum(a) * oh
                        plsc.store_scatter(
                            nout_v, [bi, r * NEG + jg * L + lane], acc)
                        return c2

                    lax.fori_loop(0, JG, gbody, 0)

                for c in out_copies(gg, b):
                    c.start()

                @pl.when(gg + 2 < NCH)
                def _():
                    fire(gg + 2, b)

            @pl.when(t % 2 == 1)
            def _():
                o = (t // 2) * L + lane
                plsc.store_scatter(pout_all, [o // 128, o % 128], pos_acc)

            return jnp.where(t % 2 == 1, jnp.zeros((L,), jnp.float32),
                             pos_acc)

        lax.fori_loop(0, NCH // 2, tbody, jnp.zeros((L,), jnp.float32))
        for c in out_copies(NCH - 2, 0):
            c.wait()
        for c in out_copies(NCH - 1, 1):
            c.wait()
        pltpu.sync_copy(pout_all, pos_hbm.at[wid])

    return pl.kernel(
        body,
        out_type=[
            jax.ShapeDtypeStruct((NW, RW // 128, 128), jnp.float32),
            jax.ShapeDtypeStruct((N * NEG,), jnp.float32),
        ],
        mesh=mesh,
        compiler_params=pltpu.CompilerParams(
            needs_layout_passes=False, use_tc_tiling_on_sc=True),
        scratch_types=[
            pltpu.VMEM((RW * IW,), jnp.int32),
            pltpu.VMEM((2, R, IW, 2 * D), jnp.float32),
            pltpu.VMEM((2, R * D), jnp.float32),
            pltpu.VMEM((2, R * NEG), jnp.float32),
            pltpu.VMEM((RW // 128, 128), jnp.float32),
            pltpu.SemaphoreType.DMA,
            pltpu.SemaphoreType.DMA,
            pltpu.SemaphoreType.DMA,
            pltpu.SemaphoreType.DMA,
        ],
    )


def _make_loss(N, NEG):
    rows = 2048
    inv_t = 1.0 / _TEMP

    def body(pos_ref, neg_ref, logits_ref, loss_ref):
        i = pl.program_id(0)
        pos = pos_ref[...]
        neg = neg_ref[...]
        logits_ref[...] = jnp.concatenate([pos, neg], axis=1)
        sp = pos * inv_t
        sn = neg * inv_t
        m = jnp.maximum(sp, jnp.max(sn, axis=1, keepdims=True))
        lse = m + jnp.log(
            jnp.exp(sp - m) + jnp.sum(jnp.exp(sn - m), axis=1, keepdims=True))
        c = jnp.sum(lse - sp)

        @pl.when(i == 0)
        def _():
            loss_ref[0, 0] = 0.0

        loss_ref[0, 0] += c / N

    return pl.pallas_call(
        body,
        grid=(N // rows,),
        in_specs=[
            pl.BlockSpec((rows, 1), lambda i: (i, 0)),
            pl.BlockSpec((rows, NEG), lambda i: (i, 0)),
        ],
        out_specs=[
            pl.BlockSpec((rows, 1 + NEG), lambda i: (i, 0)),
            pl.BlockSpec((1, 1), lambda i: (0, 0), memory_space=pltpu.SMEM),
        ],
        out_shape=[
            jax.ShapeDtypeStruct((N, 1 + NEG), jnp.float32),
            jax.ShapeDtypeStruct((1, 1), jnp.float32),
        ],
    )


def kernel(embeds, labels, table, W, b, neg_samples):
    B, T, D = embeds.shape
    N = B * T
    V = table.shape[0]
    NEG = neg_samples.shape[1]

    e2 = embeds.reshape(N, D)
    x = _project(e2, W.T, b.reshape(1, D), N, D)
    table_w = _widen_table(table.T, V, D)
    idxc = jnp.concatenate(
        [labels.reshape(N, 1), neg_samples,
         jnp.zeros((N, 7), jnp.int32)], axis=1).reshape(-1)
    pos3, neg1 = _make_scores(N, D, NEG)(x.reshape(-1), idxc, table_w)
    pos = pos3.reshape(N, 1)
    neg = neg1.reshape(N, NEG)
    logits, loss = _make_loss(N, NEG)(pos, neg)
    return logits, loss.reshape(())
